# baseline (speedup 1.0000x reference)
import functools

import jax
import jax.numpy as jnp
from jax import lax
from jax.experimental import pallas as pl
from jax.experimental.pallas import tpu as pltpu
from jax.experimental.pallas import tpu_sc as plsc

NUM_EXP = 64
CAP = 1024
N_TOK = 32768
HID = 1024
TBLK = 1024
NBLK = N_TOK // TBLK
SMAX = 2048
SSTR = 2048
KEEP_ALL = -1
DROP_ALL = 0x7F000000
IDX_PAD = 0x7FFFFFF


def _router_body(hs_ref, w_ref, e1_ref, e2_ref, p1_ref, p2_ref):
    hs = hs_ref[...]
    w = w_ref[...]
    logits = lax.dot_general(w, hs, (((1,), (1,)), ((), ())),
                             preferred_element_type=jnp.float32)
    m = jnp.max(logits, axis=0, keepdims=True)
    ex = jnp.exp(logits - m)
    z = jnp.sum(ex, axis=0, keepdims=True)
    probs = ex / z
    pmax = jnp.max(probs, axis=0, keepdims=True)
    iota = lax.broadcasted_iota(jnp.int32, (NUM_EXP, TBLK), 0)
    e1 = jnp.min(jnp.where(probs == pmax, iota, NUM_EXP), axis=0,
                 keepdims=True)
    oh1 = iota == e1
    lm = jnp.where(oh1, jnp.float32(-jnp.inf), logits)
    m2 = jnp.max(lm, axis=0, keepdims=True)
    e2 = jnp.min(jnp.where(lm == m2, iota, NUM_EXP), axis=0, keepdims=True)
    p2 = jnp.sum(jnp.where(iota == e2, probs, 0.0), axis=0, keepdims=True)
    e1_ref[...] = e1[None]
    e2_ref[...] = e2[None]
    p1_ref[...] = pmax[None]
    p2_ref[...] = p2[None]


def _run_router(hs2d, w):
    out_shape = [
        jax.ShapeDtypeStruct((NBLK, 1, TBLK), jnp.int32),
        jax.ShapeDtypeStruct((NBLK, 1, TBLK), jnp.int32),
        jax.ShapeDtypeStruct((NBLK, 1, TBLK), jnp.float32),
        jax.ShapeDtypeStruct((NBLK, 1, TBLK), jnp.float32),
    ]
    ospec = pl.BlockSpec((1, 1, TBLK), lambda i: (i, 0, 0))
    return pl.pallas_call(
        _router_body,
        grid=(NBLK,),
        in_specs=[pl.BlockSpec((TBLK, HID), lambda i: (i, 0)),
                  pl.BlockSpec((NUM_EXP, HID), lambda i: (0, 0))],
        out_specs=[ospec, ospec, ospec, ospec],
        out_shape=out_shape,
        compiler_params=pltpu.CompilerParams(
            dimension_semantics=("arbitrary",)),
    )(hs2d, w)


def _count_gt(bk_i32, base, nv, mid):
    def body(v, c):
        kb = bk_i32[pl.ds(base + v * 16, 16)]
        return c + jnp.sum((kb > mid).astype(jnp.int32))
    return lax.fori_loop(0, nv, body, jnp.int32(0))


def _select_threshold(bk, bi, base, n, t):
    nv = (n + 15) // 16

    def cnt(mid):
        def body(v, c):
            kb = bk[pl.ds(base + v * 16, 16)]
            return c + jnp.sum((kb > mid).astype(jnp.int32))
        return lax.fori_loop(0, nv, body, jnp.int32(0))

    def bis_body(_, lohi):
        lo, hi = lohi
        mid = (lo + hi) // 2
        below = cnt(mid) < t
        nlo = jnp.where(below, lo, mid + 1)
        nhi = jnp.where(below, mid, hi)
        live = lo < hi
        return (jnp.where(live, nlo, lo), jnp.where(live, nhi, hi))

    lo, _ = lax.fori_loop(0, 31, bis_body,
                          (jnp.int32(0), jnp.int32(0x3F800001)))
    vstar = lo
    c_gt = cnt(vstar)
    r = t - c_gt

    def cnt_idx(mid):
        def body(v, c):
            kb = bk[pl.ds(base + v * 16, 16)]
            ib = bi[pl.ds(base + v * 16, 16)]
            hit = (kb == vstar) & (ib <= mid)
            return c + jnp.sum(hit.astype(jnp.int32))
        return lax.fori_loop(0, nv, body, jnp.int32(0))

    def ibis_body(_, lohi):
        lo2, hi2 = lohi
        mid = (lo2 + hi2) // 2
        enough = cnt_idx(mid) >= r
        nlo = jnp.where(enough, lo2, mid + 1)
        nhi = jnp.where(enough, mid, hi2)
        live = lo2 < hi2
        return (jnp.where(live, nlo, lo2), jnp.where(live, nhi, hi2))

    ith, _ = lax.fori_loop(0, 16, ibis_body,
                           (jnp.int32(0), jnp.int32(N_TOK - 1)))
    return vstar, ith


def _sc_capacity_make():
    mesh = plsc.VectorSubcoreMesh(core_axis_name="c", subcore_axis_name="s")
    tbl = jax.ShapeDtypeStruct((32, 16), jnp.int32)
    tbl_i = jax.ShapeDtypeStruct((32, 16), jnp.int32)

    @functools.partial(
        pl.kernel,
        out_type=(tbl, tbl_i, tbl, tbl_i),
        mesh=mesh,
        compiler_params=pltpu.CompilerParams(needs_layout_passes=False),
        scratch_types=[
            pltpu.VMEM((N_TOK,), jnp.int32),
            pltpu.VMEM((N_TOK,), jnp.int32),
            pltpu.VMEM((N_TOK,), jnp.float32),
            pltpu.VMEM((SSTR + 64,), jnp.int32),
            pltpu.VMEM((SSTR + 64,), jnp.int32),
            pltpu.VMEM((SSTR + 64,), jnp.int32),
            pltpu.VMEM((SSTR + 64,), jnp.int32),
            pltpu.VMEM((4 * SMAX + 64,), jnp.int32),
            pltpu.VMEM((4 * SMAX + 64,), jnp.int32),
            pltpu.VMEM((16,), jnp.int32),
            pltpu.VMEM((16,), jnp.int32),
            pltpu.VMEM((16,), jnp.int32),
            pltpu.VMEM((16,), jnp.int32),
        ],
    )
    def _sc_capacity(e1_hbm, e2_hbm, key_hbm,
                     kth1_hbm, ith1_hbm, kth2_hbm, ith2_hbm,
                     e1_v, e2_v, key_v, sk1, sp1, sk2, sp2,
                     bk, bi, ob1k, ob1i, ob2k, ob2i):
        wid = lax.axis_index("s") * 2 + lax.axis_index("c")
        pltpu.sync_copy(e1_hbm, e1_v)
        pltpu.sync_copy(e2_hbm, e2_v)
        pltpu.sync_copy(key_hbm, key_v)

        lane16 = jnp.arange(16, dtype=jnp.int32)

        def pre(v, _):
            for u in range(4):
                bk[pl.ds((v * 4 + u) * 16, 16)] = \
                    jnp.full((16,), -1, jnp.int32)
                bi[pl.ds((v * 4 + u) * 16, 16)] = \
                    jnp.full((16,), IDX_PAD, jnp.int32)
            return 0
        lax.fori_loop(0, SMAX // 16 + 1, pre, 0)

        def pre2(v, _):
            for u in range(4):
                sp1[pl.ds((v * 4 + u) * 16, 16)] = \
                    jnp.full((16,), IDX_PAD, jnp.int32)
                sp2[pl.ds((v * 4 + u) * 16, 16)] = \
                    jnp.full((16,), IDX_PAD, jnp.int32)
            return 0
        lax.fori_loop(0, SSTR // 16 // 4 + 1, pre2, 0)

        UNR = 8

        def scan(v, offs):
            o1, o2 = offs
            for u in range(UNR):
                base = (v * UNR + u) * 16
                kv = plsc.bitcast(key_v[pl.ds(base, 16)], jnp.int32)
                e1v = e1_v[pl.ds(base, 16)]
                e2v = e2_v[pl.ds(base, 16)]
                idxv = lane16 + base
                m1 = (e1v >> 1) == wid
                d1 = jnp.where(m1,
                               jnp.minimum(o1, SSTR - 16)
                               + plsc.cumsum(m1.astype(jnp.int32)) - 1,
                               SSTR + lane16)
                plsc.store_scatter(sk1, [d1], kv)
                plsc.store_scatter(sp1, [d1], idxv + ((e1v & 1) << 15))
                m2 = (e2v >> 1) == wid
                d2 = jnp.where(m2,
                               jnp.minimum(o2, SSTR - 16)
                               + plsc.cumsum(m2.astype(jnp.int32)) - 1,
                               SSTR + lane16)
                plsc.store_scatter(sk2, [d2], kv)
                plsc.store_scatter(sp2, [d2], idxv + ((e2v & 1) << 15))
                o1 = o1 + jnp.sum(m1.astype(jnp.int32))
                o2 = o2 + jnp.sum(m2.astype(jnp.int32))
            return (o1, o2)

        z = jnp.int32(0)
        o1, o2 = lax.fori_loop(0, N_TOK // 16 // UNR, scan, (z, z))

        def split(sk, sp, o, grp):
            nv = (jnp.minimum(o, SSTR) + 15) // 16

            def body(v, offs):
                res = list(offs)
                kb = sk[pl.ds(v * 16, 16)]
                pkv = sp[pl.ds(v * 16, 16)]
                bit = pkv >> 15
                idx = pkv & 0x7FFF
                for t in range(2):
                    m = bit == t
                    mi = m.astype(jnp.int32)
                    wo = ((grp * 2 + t) * SMAX
                          + jnp.minimum(res[t], SMAX - 16))
                    d = jnp.where(m, wo + plsc.cumsum(mi) - 1,
                                  4 * SMAX + lane16)
                    plsc.store_scatter(bk, [d], kb)
                    plsc.store_scatter(bi, [d], idx)
                    res[t] = res[t] + jnp.sum(mi)
                return tuple(res)

            return lax.fori_loop(0, nv, body, (z, z))

        c1a, c1b = split(sk1, sp1, o1, 0)
        c2a, c2b = split(sk2, sp2, o2, 1)

        def thresholds(t, cnt1, cnt2):
            slot1, slot2 = t, 2 + t
            n1 = jnp.minimum(cnt1, SMAX)
            n2 = jnp.minimum(cnt2, SMAX)
            k1, i1 = lax.cond(
                cnt1 <= CAP,
                lambda: (jnp.int32(KEEP_ALL), jnp.int32(0)),
                lambda: _select_threshold(bk, bi, slot1 * SMAX, n1,
                                          jnp.int32(CAP)))
            t2 = CAP - cnt1
            k2, i2 = lax.cond(
                t2 <= 0,
                lambda: (jnp.int32(DROP_ALL), jnp.int32(0)),
                lambda: lax.cond(
                    t2 >= cnt2,
                    lambda: (jnp.int32(KEEP_ALL), jnp.int32(0)),
                    lambda: _select_threshold(bk, bi, slot2 * SMAX, n2, t2)))
            return k1, i1, k2, i2

        k1a, i1a, k2a, i2a = thresholds(0, c1a, c2a)
        k1b, i1b, k2b, i2b = thresholds(1, c1b, c2b)

        lane = jnp.arange(16, dtype=jnp.int32)
        ob1k[...] = jnp.where(lane == 0, k1a, jnp.where(lane == 1, k1b, 0))
        ob1i[...] = jnp.where(lane == 0, i1a, jnp.where(lane == 1, i1b, 0))
        ob2k[...] = jnp.where(lane == 0, k2a, jnp.where(lane == 1, k2b, 0))
        ob2i[...] = jnp.where(lane == 0, i2a, jnp.where(lane == 1, i2b, 0))
        pltpu.sync_copy(ob1k, kth1_hbm.at[wid])
        pltpu.sync_copy(ob1i, ith1_hbm.at[wid])
        pltpu.sync_copy(ob2k, kth2_hbm.at[wid])
        pltpu.sync_copy(ob2i, ith2_hbm.at[wid])

    return _sc_capacity


CBLK = TBLK


def _finalize_body(e1_ref, e2_ref, p1_ref, p2_ref,
                   k1_ref, i1_ref, k2_ref, i2_ref,
                   mask_ref, comb_ref):
    blk = pl.program_id(0)
    e1 = e1_ref[0]
    e2 = e2_ref[0]
    p1 = p1_ref[0]
    p2 = p2_ref[0]
    p1b = lax.bitcast_convert_type(p1, jnp.int32)
    iota_e = lax.broadcasted_iota(jnp.int32, (NUM_EXP, CBLK), 0)
    oh1 = iota_e == e1
    oh2 = iota_e == e2
    k1t = k1_ref[:, 0:1]
    i1t = i1_ref[:, 0:1]
    k2t = k2_ref[:, 0:1]
    i2t = i2_ref[:, 0:1]
    k1g = jnp.sum(jnp.where(oh1, k1t, 0), axis=0, keepdims=True)
    i1g = jnp.sum(jnp.where(oh1, i1t, 0), axis=0, keepdims=True)
    k2g = jnp.sum(jnp.where(oh2, k2t, 0), axis=0, keepdims=True)
    i2g = jnp.sum(jnp.where(oh2, i2t, 0), axis=0, keepdims=True)
    tok = (lax.broadcasted_iota(jnp.int32, (1, CBLK), 1) + blk * CBLK)
    keep1 = (p1b > k1g) | ((p1b == k1g) & (tok <= i1g))
    keep2 = (p1b > k2g) | ((p1b == k2g) & (tok <= i2g))
    s1 = jnp.where(keep1, p1, 0.0)
    s2 = jnp.where(keep2, p2, 0.0)
    denom = jnp.maximum(s1 + s2, jnp.float32(jnp.finfo(jnp.float32).eps))
    g1 = s1 / denom
    g2 = s2 / denom
    mask_t = (oh1 & keep1).astype(jnp.int32)
    comb_t = jnp.where(oh1 & keep1, g1, 0.0) + \
        jnp.where(oh2 & keep2, g2, 0.0)
    mask_ref[...] = jnp.transpose(mask_t)
    comb_ref[...] = jnp.transpose(comb_t)


def _run_finalize(e1r, e2r, p1r, p2r, k1, i1, k2, i2):
    nblk = N_TOK // CBLK
    ispec = pl.BlockSpec((1, 1, CBLK), lambda i: (i, 0, 0))
    tspec = pl.BlockSpec((NUM_EXP, 128), lambda i: (0, 0))
    ospec = pl.BlockSpec((CBLK, NUM_EXP), lambda i: (i, 0))
    return pl.pallas_call(
        _finalize_body,
        grid=(nblk,),
        in_specs=[ispec, ispec, ispec, ispec,
                  tspec, tspec, tspec, tspec],
        out_specs=[ospec, ospec],
        out_shape=[jax.ShapeDtypeStruct((N_TOK, NUM_EXP), jnp.int32),
                   jax.ShapeDtypeStruct((N_TOK, NUM_EXP), jnp.float32)],
        compiler_params=pltpu.CompilerParams(
            dimension_semantics=("arbitrary",)),
    )(e1r, e2r, p1r, p2r, k1, i1, k2, i2)


def kernel(hidden_states, W):
    b, s, h = hidden_states.shape
    hs2d = hidden_states.reshape(b * s, h).astype(jnp.float32)
    e1r, e2r, p1r, p2r = _run_router(hs2d, W)

    sc = _sc_capacity_make()
    kth1, ith1, kth2, ith2 = sc(e1r.reshape(N_TOK), e2r.reshape(N_TOK),
                                p1r.reshape(N_TOK))

    def tab(x):
        return jnp.broadcast_to(x[:, :2].reshape(NUM_EXP, 1), (NUM_EXP, 128))

    mask1, comb = _run_finalize(
        e1r, e2r, p1r, p2r,
        tab(kth1), tab(ith1), tab(kth2), tab(ith2))
    return (mask1, comb)


# speedup vs baseline: 1.3501x; 1.3501x over previous
import functools

import jax
import jax.numpy as jnp
from jax import lax
from jax.experimental import pallas as pl
from jax.experimental.pallas import tpu as pltpu
from jax.experimental.pallas import tpu_sc as plsc

NUM_EXP = 64
CAP = 1024
N_TOK = 32768
HID = 1024
TBLK = 1024
NBLK = N_TOK // TBLK
SMAX = 2048
SSTR = 2048
KEEP_ALL = -1
DROP_ALL = 0x7F000000
IDX_PAD = 0x7FFFFFF


def _router_body(hs_ref, w_ref, e1_ref, e2_ref, p1_ref, p2_ref):
    hs = hs_ref[...]
    w = w_ref[...]
    logits = lax.dot_general(w, hs, (((1,), (1,)), ((), ())),
                             preferred_element_type=jnp.float32)
    m = jnp.max(logits, axis=0, keepdims=True)
    ex = jnp.exp(logits - m)
    z = jnp.sum(ex, axis=0, keepdims=True)
    probs = ex / z
    pmax = jnp.max(probs, axis=0, keepdims=True)
    iota = lax.broadcasted_iota(jnp.int32, (NUM_EXP, TBLK), 0)
    e1 = jnp.min(jnp.where(probs == pmax, iota, NUM_EXP), axis=0,
                 keepdims=True)
    oh1 = iota == e1
    lm = jnp.where(oh1, jnp.float32(-jnp.inf), logits)
    m2 = jnp.max(lm, axis=0, keepdims=True)
    e2 = jnp.min(jnp.where(lm == m2, iota, NUM_EXP), axis=0, keepdims=True)
    p2 = jnp.sum(jnp.where(iota == e2, probs, 0.0), axis=0, keepdims=True)
    e1_ref[...] = e1[None]
    e2_ref[...] = e2[None]
    p1_ref[...] = pmax[None]
    p2_ref[...] = p2[None]


def _run_router(hs2d, w):
    out_shape = [
        jax.ShapeDtypeStruct((NBLK, 1, TBLK), jnp.int32),
        jax.ShapeDtypeStruct((NBLK, 1, TBLK), jnp.int32),
        jax.ShapeDtypeStruct((NBLK, 1, TBLK), jnp.float32),
        jax.ShapeDtypeStruct((NBLK, 1, TBLK), jnp.float32),
    ]
    ospec = pl.BlockSpec((1, 1, TBLK), lambda i: (i, 0, 0))
    return pl.pallas_call(
        _router_body,
        grid=(NBLK,),
        in_specs=[pl.BlockSpec((TBLK, HID), lambda i: (i, 0)),
                  pl.BlockSpec((NUM_EXP, HID), lambda i: (0, 0))],
        out_specs=[ospec, ospec, ospec, ospec],
        out_shape=out_shape,
        compiler_params=pltpu.CompilerParams(
            dimension_semantics=("arbitrary",)),
    )(hs2d, w)


def _count_gt(bk_i32, base, nv, mid):
    def body(v, c):
        kb = bk_i32[pl.ds(base + v * 16, 16)]
        return c + jnp.sum((kb > mid).astype(jnp.int32))
    return lax.fori_loop(0, nv, body, jnp.int32(0))


def _select_threshold(bk, bi, base, n, t):
    nv = (n + 15) // 16

    def cnt(mid):
        def body(v, c):
            kb = bk[pl.ds(base + v * 16, 16)]
            return c + jnp.sum((kb > mid).astype(jnp.int32))
        return lax.fori_loop(0, nv, body, jnp.int32(0))

    def bis_body(_, lohi):
        lo, hi = lohi
        mid = (lo + hi) // 2
        below = cnt(mid) < t
        nlo = jnp.where(below, lo, mid + 1)
        nhi = jnp.where(below, mid, hi)
        live = lo < hi
        return (jnp.where(live, nlo, lo), jnp.where(live, nhi, hi))

    lo, _ = lax.fori_loop(0, 31, bis_body,
                          (jnp.int32(0), jnp.int32(0x3F800001)))
    vstar = lo
    c_gt = cnt(vstar)
    r = t - c_gt

    def cnt_idx(mid):
        def body(v, c):
            kb = bk[pl.ds(base + v * 16, 16)]
            ib = bi[pl.ds(base + v * 16, 16)]
            hit = (kb == vstar) & (ib <= mid)
            return c + jnp.sum(hit.astype(jnp.int32))
        return lax.fori_loop(0, nv, body, jnp.int32(0))

    def ibis_body(_, lohi):
        lo2, hi2 = lohi
        mid = (lo2 + hi2) // 2
        enough = cnt_idx(mid) >= r
        nlo = jnp.where(enough, lo2, mid + 1)
        nhi = jnp.where(enough, mid, hi2)
        live = lo2 < hi2
        return (jnp.where(live, nlo, lo2), jnp.where(live, nhi, hi2))

    ith, _ = lax.fori_loop(0, 16, ibis_body,
                           (jnp.int32(0), jnp.int32(N_TOK - 1)))
    return vstar, ith


def _sc_capacity_make():
    mesh = plsc.VectorSubcoreMesh(core_axis_name="c", subcore_axis_name="s")
    tbl = jax.ShapeDtypeStruct((32, 16), jnp.int32)
    tbl_i = jax.ShapeDtypeStruct((32, 16), jnp.int32)

    @functools.partial(
        pl.kernel,
        out_type=(tbl, tbl_i, tbl, tbl_i),
        mesh=mesh,
        compiler_params=pltpu.CompilerParams(needs_layout_passes=False),
        scratch_types=[
            pltpu.VMEM((N_TOK,), jnp.int32),
            pltpu.VMEM((N_TOK,), jnp.int32),
            pltpu.VMEM((N_TOK,), jnp.float32),
            pltpu.VMEM((SSTR + 64,), jnp.int32),
            pltpu.VMEM((SSTR + 64,), jnp.int32),
            pltpu.VMEM((SSTR + 64,), jnp.int32),
            pltpu.VMEM((SSTR + 64,), jnp.int32),
            pltpu.VMEM((4 * SMAX + 64,), jnp.int32),
            pltpu.VMEM((4 * SMAX + 64,), jnp.int32),
            pltpu.VMEM((16,), jnp.int32),
            pltpu.VMEM((16,), jnp.int32),
            pltpu.VMEM((16,), jnp.int32),
            pltpu.VMEM((16,), jnp.int32),
        ],
    )
    def _sc_capacity(e1_hbm, e2_hbm, key_hbm,
                     kth1_hbm, ith1_hbm, kth2_hbm, ith2_hbm,
                     e1_v, e2_v, key_v, sk1, sp1, sk2, sp2,
                     bk, bi, ob1k, ob1i, ob2k, ob2i):
        wid = lax.axis_index("s") * 2 + lax.axis_index("c")
        pltpu.sync_copy(e1_hbm, e1_v)
        pltpu.sync_copy(e2_hbm, e2_v)
        pltpu.sync_copy(key_hbm, key_v)

        lane16 = jnp.arange(16, dtype=jnp.int32)

        def pre(v, _):
            for u in range(4):
                bk[pl.ds((v * 4 + u) * 16, 16)] = \
                    jnp.full((16,), -1, jnp.int32)
                bi[pl.ds((v * 4 + u) * 16, 16)] = \
                    jnp.full((16,), IDX_PAD, jnp.int32)
            return 0
        lax.fori_loop(0, SMAX // 16 + 1, pre, 0)

        def pre2(v, _):
            for u in range(4):
                sp1[pl.ds((v * 4 + u) * 16, 16)] = \
                    jnp.full((16,), IDX_PAD, jnp.int32)
                sp2[pl.ds((v * 4 + u) * 16, 16)] = \
                    jnp.full((16,), IDX_PAD, jnp.int32)
            return 0
        lax.fori_loop(0, SSTR // 16 // 4 + 1, pre2, 0)

        UNR = 8

        def scan(v, offs):
            o1, o2 = offs
            o1c = jnp.minimum(o1, SSTR - 160)
            o2c = jnp.minimum(o2, SSTR - 160)
            cums1, cums2, kvs, pk1s, pk2s = [], [], [], [], []
            m1s, m2s = [], []
            for u in range(UNR):
                base = (v * UNR + u) * 16
                kv = plsc.bitcast(key_v[pl.ds(base, 16)], jnp.int32)
                e1v = e1_v[pl.ds(base, 16)]
                e2v = e2_v[pl.ds(base, 16)]
                idxv = lane16 + base
                m1 = (e1v >> 1) == wid
                m2 = (e2v >> 1) == wid
                cums1.append(plsc.cumsum(m1.astype(jnp.int32)))
                cums2.append(plsc.cumsum(m2.astype(jnp.int32)))
                kvs.append(kv)
                m1s.append(m1)
                m2s.append(m2)
                pk1s.append(idxv + ((e1v & 1) << 15))
                pk2s.append(idxv + ((e2v & 1) << 15))
            r1 = jnp.int32(0)
            r2 = jnp.int32(0)
            for u in range(UNR):
                cs1, cs2 = cums1[u], cums2[u]
                d1 = jnp.where(m1s[u], o1c + r1 + cs1 - 1, SSTR + lane16)
                plsc.store_scatter(sk1, [d1], kvs[u])
                plsc.store_scatter(sp1, [d1], pk1s[u])
                r1 = r1 + cs1[15]
                d2 = jnp.where(m2s[u], o2c + r2 + cs2 - 1, SSTR + lane16)
                plsc.store_scatter(sk2, [d2], kvs[u])
                plsc.store_scatter(sp2, [d2], pk2s[u])
                r2 = r2 + cs2[15]
            return (o1 + r1, o2 + r2)

        z = jnp.int32(0)
        o1, o2 = lax.fori_loop(0, N_TOK // 16 // UNR, scan, (z, z))

        def split(sk, sp, o, grp):
            nv = (jnp.minimum(o, SSTR) + 15) // 16

            def body(v, offs):
                res = list(offs)
                kb = sk[pl.ds(v * 16, 16)]
                pkv = sp[pl.ds(v * 16, 16)]
                bit = pkv >> 15
                idx = pkv & 0x7FFF
                for t in range(2):
                    m = bit == t
                    mi = m.astype(jnp.int32)
                    wo = ((grp * 2 + t) * SMAX
                          + jnp.minimum(res[t], SMAX - 16))
                    d = jnp.where(m, wo + plsc.cumsum(mi) - 1,
                                  4 * SMAX + lane16)
                    plsc.store_scatter(bk, [d], kb)
                    plsc.store_scatter(bi, [d], idx)
                    res[t] = res[t] + jnp.sum(mi)
                return tuple(res)

            return lax.fori_loop(0, nv, body, (z, z))

        c1a, c1b = split(sk1, sp1, o1, 0)
        c2a, c2b = split(sk2, sp2, o2, 1)

        def thresholds(t, cnt1, cnt2):
            slot1, slot2 = t, 2 + t
            n1 = jnp.minimum(cnt1, SMAX)
            n2 = jnp.minimum(cnt2, SMAX)
            k1, i1 = lax.cond(
                cnt1 <= CAP,
                lambda: (jnp.int32(KEEP_ALL), jnp.int32(0)),
                lambda: _select_threshold(bk, bi, slot1 * SMAX, n1,
                                          jnp.int32(CAP)))
            t2 = CAP - cnt1
            k2, i2 = lax.cond(
                t2 <= 0,
                lambda: (jnp.int32(DROP_ALL), jnp.int32(0)),
                lambda: lax.cond(
                    t2 >= cnt2,
                    lambda: (jnp.int32(KEEP_ALL), jnp.int32(0)),
                    lambda: _select_threshold(bk, bi, slot2 * SMAX, n2, t2)))
            return k1, i1, k2, i2

        k1a, i1a, k2a, i2a = thresholds(0, c1a, c2a)
        k1b, i1b, k2b, i2b = thresholds(1, c1b, c2b)

        lane = jnp.arange(16, dtype=jnp.int32)
        ob1k[...] = jnp.where(lane == 0, k1a, jnp.where(lane == 1, k1b, 0))
        ob1i[...] = jnp.where(lane == 0, i1a, jnp.where(lane == 1, i1b, 0))
        ob2k[...] = jnp.where(lane == 0, k2a, jnp.where(lane == 1, k2b, 0))
        ob2i[...] = jnp.where(lane == 0, i2a, jnp.where(lane == 1, i2b, 0))
        pltpu.sync_copy(ob1k, kth1_hbm.at[wid])
        pltpu.sync_copy(ob1i, ith1_hbm.at[wid])
        pltpu.sync_copy(ob2k, kth2_hbm.at[wid])
        pltpu.sync_copy(ob2i, ith2_hbm.at[wid])

    return _sc_capacity


CBLK = TBLK


def _finalize_body(e1_ref, e2_ref, p1_ref, p2_ref,
                   k1_ref, i1_ref, k2_ref, i2_ref,
                   mask_ref, comb_ref):
    blk = pl.program_id(0)
    e1 = e1_ref[0]
    e2 = e2_ref[0]
    p1 = p1_ref[0]
    p2 = p2_ref[0]
    p1b = lax.bitcast_convert_type(p1, jnp.int32)
    iota_e = lax.broadcasted_iota(jnp.int32, (NUM_EXP, CBLK), 0)
    oh1 = iota_e == e1
    oh2 = iota_e == e2
    k1t = k1_ref[:, 0:1]
    i1t = i1_ref[:, 0:1]
    k2t = k2_ref[:, 0:1]
    i2t = i2_ref[:, 0:1]
    k1g = jnp.sum(jnp.where(oh1, k1t, 0), axis=0, keepdims=True)
    i1g = jnp.sum(jnp.where(oh1, i1t, 0), axis=0, keepdims=True)
    k2g = jnp.sum(jnp.where(oh2, k2t, 0), axis=0, keepdims=True)
    i2g = jnp.sum(jnp.where(oh2, i2t, 0), axis=0, keepdims=True)
    tok = (lax.broadcasted_iota(jnp.int32, (1, CBLK), 1) + blk * CBLK)
    keep1 = (p1b > k1g) | ((p1b == k1g) & (tok <= i1g))
    keep2 = (p1b > k2g) | ((p1b == k2g) & (tok <= i2g))
    s1 = jnp.where(keep1, p1, 0.0)
    s2 = jnp.where(keep2, p2, 0.0)
    denom = jnp.maximum(s1 + s2, jnp.float32(jnp.finfo(jnp.float32).eps))
    g1 = s1 / denom
    g2 = s2 / denom
    mask_ref[...] = (oh1 & keep1).astype(jnp.int32)
    comb_ref[...] = jnp.where(oh1 & keep1, g1, 0.0) + \
        jnp.where(oh2 & keep2, g2, 0.0)


def _run_finalize(e1r, e2r, p1r, p2r, k1, i1, k2, i2):
    nblk = N_TOK // CBLK
    ispec = pl.BlockSpec((1, 1, CBLK), lambda i: (i, 0, 0))
    tspec = pl.BlockSpec((NUM_EXP, 128), lambda i: (0, 0))
    ospec = pl.BlockSpec((NUM_EXP, CBLK), lambda i: (0, i))
    return pl.pallas_call(
        _finalize_body,
        grid=(nblk,),
        in_specs=[ispec, ispec, ispec, ispec,
                  tspec, tspec, tspec, tspec],
        out_specs=[ospec, ospec],
        out_shape=[jax.ShapeDtypeStruct((NUM_EXP, N_TOK), jnp.int32),
                   jax.ShapeDtypeStruct((NUM_EXP, N_TOK), jnp.float32)],
        compiler_params=pltpu.CompilerParams(
            dimension_semantics=("arbitrary",)),
    )(e1r, e2r, p1r, p2r, k1, i1, k2, i2)


def kernel(hidden_states, W):
    b, s, h = hidden_states.shape
    hs2d = hidden_states.reshape(b * s, h).astype(jnp.float32)
    e1r, e2r, p1r, p2r = _run_router(hs2d, W)

    sc = _sc_capacity_make()
    kth1, ith1, kth2, ith2 = sc(e1r.reshape(N_TOK), e2r.reshape(N_TOK),
                                p1r.reshape(N_TOK))

    def tab(x):
        return jnp.broadcast_to(x[:, :2].reshape(NUM_EXP, 1), (NUM_EXP, 128))

    mask_t, comb_t = _run_finalize(
        e1r, e2r, p1r, p2r,
        tab(kth1), tab(ith1), tab(kth2), tab(ith2))
    return (mask_t.T, comb_t.T)
